# baseline (speedup 1.0000x reference)
import jax
import jax.numpy as jnp
from jax.experimental import pallas as pl
from jax.experimental.pallas import tpu as pltpu

B, N, M, V = 128, 8, 64, 512
K = 8
T = 30
PAD = 0
NEG_INF = -1e30
G = 16


def _one_sample(logits, g, span_b, tt_b, dec):
    col = jax.lax.broadcasted_iota(jnp.int32, (K, N + 1), 1)
    masked = jnp.where(col <= span_b, logits, NEG_INF)
    shifted = masked - jnp.max(masked, axis=1, keepdims=True)
    logp = shifted - jnp.log(jnp.sum(jnp.exp(shifted), axis=1, keepdims=True))
    z = logp + g
    ez = jnp.exp(z - jnp.max(z, axis=1, keepdims=True))
    soft = ez / jnp.sum(ez, axis=1, keepdims=True)
    sel = jnp.argmax(soft, axis=1).reshape(K, 1).astype(jnp.int32)
    krow = jax.lax.broadcasted_iota(jnp.int32, (K, 1), 0)
    sel = jnp.where(tt_b == 20, jnp.where(krow == 0, 1, 0), sel)

    rowmax = jnp.max(dec, axis=2)
    col0 = dec[:, :, 0]
    nsel = jnp.broadcast_to(jnp.maximum(sel - 1, 0), (K, M))
    rowmax_sel = jnp.take_along_axis(rowmax, nsel, axis=0)
    col0_sel = jnp.take_along_axis(col0, nsel, axis=0)
    midx = jax.lax.broadcasted_iota(jnp.int32, (K, M), 1)
    nz = (col0_sel < rowmax_sel) & (sel > 0)
    lens = jnp.max(jnp.where(nz, midx + 1, 0), axis=1).reshape(K, 1)

    kk_r = jax.lax.broadcasted_iota(jnp.int32, (K, K), 0)
    kk_c = jax.lax.broadcasted_iota(jnp.int32, (K, K), 1)
    lens_row = lens.reshape(1, K)
    excl = jnp.sum(jnp.where(kk_c < kk_r, jnp.broadcast_to(lens_row, (K, K)), 0),
                   axis=1).reshape(K, 1)
    idx = jnp.minimum(excl, M)
    olen = jnp.minimum(lens, M - idx)

    j = jax.lax.broadcasted_iota(jnp.int32, (K, M), 1)
    in_seg = (j >= idx) & (j < idx + olen)
    srcval = (sel - 1) * M + (j - idx)
    src = jnp.sum(jnp.where(in_seg, srcval, 0), axis=0).reshape(M, 1)
    covered = jnp.sum(in_seg.astype(jnp.int32), axis=0).reshape(M, 1) > 0
    src = jnp.where(covered, src, -1)

    i_flat = jax.lax.broadcasted_iota(jnp.int32, (M, N * M), 1)
    P = (i_flat == src).astype(jnp.float32)
    dec_flat = dec.reshape(N * M, V)
    return jax.lax.dot(P, dec_flat, preferred_element_type=jnp.float32)


def _kernel(spans_ref, tt_ref, logits_ref, gumbel_ref, dec_ref, out_ref):
    i = pl.program_id(0)
    for gi in range(G):
        b = i * G + gi
        span_b = spans_ref[b]
        tt_b = tt_ref[b]
        logits = logits_ref[span_b - 2, tt_b - 9]
        gum = gumbel_ref[gi]
        dec = dec_ref[gi]
        out_ref[gi] = _one_sample(logits, gum, span_b, tt_b, dec)


def kernel(decodings, variables, dec_sem_logits, gumbel_noise, target_types, spans):
    del variables
    spans = spans.astype(jnp.int32)
    target_types = target_types.astype(jnp.int32)

    grid_spec = pltpu.PrefetchScalarGridSpec(
        num_scalar_prefetch=2,
        grid=(B // G,),
        in_specs=[
            pl.BlockSpec((N - 1, T - 9, K, N + 1), lambda i, *_: (0, 0, 0, 0)),
            pl.BlockSpec((G, K, N + 1), lambda i, *_: (i, 0, 0)),
            pl.BlockSpec((G, N, M, V), lambda i, *_: (i, 0, 0, 0)),
        ],
        out_specs=pl.BlockSpec((G, M, V), lambda i, *_: (i, 0, 0)),
    )
    return pl.pallas_call(
        _kernel,
        out_shape=jax.ShapeDtypeStruct((B, M, V), jnp.float32),
        grid_spec=grid_spec,
        compiler_params=pltpu.CompilerParams(
            dimension_semantics=("parallel",),
            vmem_limit_bytes=100 * 1024 * 1024,
        ),
        name="tree_lstm_template",
    )(spans, target_types, dec_sem_logits, gumbel_noise, decodings)


# speedup vs baseline: 1.3747x; 1.3747x over previous
import jax
import jax.numpy as jnp
from jax.experimental import pallas as pl
from jax.experimental.pallas import tpu as pltpu

B, N, M, V = 128, 8, 64, 512
K = 8
T = 30
PAD = 0
NEG_INF = -1e30
G = 4


def _one_sample(logits, g, span_b, tt_b, dec):
    col = jax.lax.broadcasted_iota(jnp.int32, (K, N + 1), 1)
    masked = jnp.where(col <= span_b, logits, NEG_INF)
    shifted = masked - jnp.max(masked, axis=1, keepdims=True)
    logp = shifted - jnp.log(jnp.sum(jnp.exp(shifted), axis=1, keepdims=True))
    z = logp + g
    ez = jnp.exp(z - jnp.max(z, axis=1, keepdims=True))
    soft = ez / jnp.sum(ez, axis=1, keepdims=True)
    sel = jnp.argmax(soft, axis=1).reshape(K, 1).astype(jnp.int32)
    krow = jax.lax.broadcasted_iota(jnp.int32, (K, 1), 0)
    sel = jnp.where(tt_b == 20, jnp.where(krow == 0, 1, 0), sel)

    rowmax = jnp.max(dec, axis=2)
    col0 = dec[:, :, 0]
    nsel = jnp.broadcast_to(jnp.maximum(sel - 1, 0), (K, M))
    rowmax_sel = jnp.take_along_axis(rowmax, nsel, axis=0)
    col0_sel = jnp.take_along_axis(col0, nsel, axis=0)
    midx = jax.lax.broadcasted_iota(jnp.int32, (K, M), 1)
    nz = (col0_sel < rowmax_sel) & (sel > 0)
    lens = jnp.max(jnp.where(nz, midx + 1, 0), axis=1).reshape(K, 1)

    kk_r = jax.lax.broadcasted_iota(jnp.int32, (K, K), 0)
    kk_c = jax.lax.broadcasted_iota(jnp.int32, (K, K), 1)
    lens_row = lens.reshape(1, K)
    excl = jnp.sum(jnp.where(kk_c < kk_r, jnp.broadcast_to(lens_row, (K, K)), 0),
                   axis=1).reshape(K, 1)
    idx = jnp.minimum(excl, M)
    olen = jnp.minimum(lens, M - idx)

    j = jax.lax.broadcasted_iota(jnp.int32, (K, M), 1)
    in_seg = (j >= idx) & (j < idx + olen)
    srcval = (sel - 1) * M + (j - idx)
    src = jnp.sum(jnp.where(in_seg, srcval, 0), axis=0).reshape(M, 1)
    covered = jnp.sum(in_seg.astype(jnp.int32), axis=0).reshape(M, 1) > 0
    src = jnp.where(covered, src, -1)

    i_flat = jax.lax.broadcasted_iota(jnp.int32, (M, N * M), 1)
    P = (i_flat == src).astype(jnp.float32)
    dec_flat = dec.reshape(N * M, V)
    return jax.lax.dot(P, dec_flat, preferred_element_type=jnp.float32)


def _kernel(spans_ref, tt_ref, logits_ref, gumbel_ref, dec_ref, out_ref):
    i = pl.program_id(0)
    for gi in range(G):
        b = i * G + gi
        span_b = spans_ref[b]
        tt_b = tt_ref[b]
        logits = logits_ref[span_b - 2, tt_b - 9]
        gum = gumbel_ref[gi]
        dec = dec_ref[gi]
        out_ref[gi] = _one_sample(logits, gum, span_b, tt_b, dec)


def kernel(decodings, variables, dec_sem_logits, gumbel_noise, target_types, spans):
    del variables
    spans = spans.astype(jnp.int32)
    target_types = target_types.astype(jnp.int32)

    grid_spec = pltpu.PrefetchScalarGridSpec(
        num_scalar_prefetch=2,
        grid=(B // G,),
        in_specs=[
            pl.BlockSpec((N - 1, T - 9, K, N + 1), lambda i, *_: (0, 0, 0, 0)),
            pl.BlockSpec((G, K, N + 1), lambda i, *_: (i, 0, 0)),
            pl.BlockSpec((G, N, M, V), lambda i, *_: (i, 0, 0, 0)),
        ],
        out_specs=pl.BlockSpec((G, M, V), lambda i, *_: (i, 0, 0)),
    )
    return pl.pallas_call(
        _kernel,
        out_shape=jax.ShapeDtypeStruct((B, M, V), jnp.float32),
        grid_spec=grid_spec,
        compiler_params=pltpu.CompilerParams(
            dimension_semantics=("parallel",),
            vmem_limit_bytes=100 * 1024 * 1024,
        ),
        name="tree_lstm_template",
    )(spans, target_types, dec_sem_logits, gumbel_noise, decodings)


# speedup vs baseline: 1.4466x; 1.0523x over previous
import jax
import jax.numpy as jnp
from jax.experimental import pallas as pl
from jax.experimental.pallas import tpu as pltpu

B, N, M, V = 128, 8, 64, 512
K = 8
T = 30
PAD = 0
NEG_INF = -1e30
G = 4
GK = G * K


def _kernel(spans_ref, tt_ref, logits_ref, gumbel_ref, dec_ref, out_ref):
    i = pl.program_id(0)

    logit_rows = []
    span_cols = []
    tt_cols = []
    for gi in range(G):
        b = i * G + gi
        span_b = spans_ref[b]
        tt_b = tt_ref[b]
        logit_rows.append(logits_ref[span_b - 2, tt_b - 9])
        span_cols.append(jnp.full((K, 1), span_b, jnp.int32))
        tt_cols.append(jnp.full((K, 1), tt_b, jnp.int32))
    logits = jnp.concatenate(logit_rows, axis=0)
    span_col = jnp.concatenate(span_cols, axis=0)
    tt_col = jnp.concatenate(tt_cols, axis=0)
    gum = gumbel_ref[:].reshape(GK, N + 1)

    col = jax.lax.broadcasted_iota(jnp.int32, (GK, N + 1), 1)
    masked = jnp.where(col <= span_col, logits, NEG_INF)
    shifted = masked - jnp.max(masked, axis=1, keepdims=True)
    logp = shifted - jnp.log(jnp.sum(jnp.exp(shifted), axis=1, keepdims=True))
    z = logp + gum
    ez = jnp.exp(z - jnp.max(z, axis=1, keepdims=True))
    soft = ez / jnp.sum(ez, axis=1, keepdims=True)
    sel = jnp.argmax(soft, axis=1).reshape(GK, 1).astype(jnp.int32)
    krow = jax.lax.broadcasted_iota(jnp.int32, (GK, 1), 0) % K
    sel = jnp.where(tt_col == 20, jnp.where(krow == 0, 1, 0), sel)

    dec_all = dec_ref[:]
    rowmax_all = jnp.max(dec_all, axis=3)
    col0_all = dec_all[:, :, :, 0]

    rm_rows = []
    c0_rows = []
    for gi in range(G):
        sel_g = sel[gi * K:(gi + 1) * K]
        nsel = jnp.broadcast_to(jnp.maximum(sel_g - 1, 0), (K, M))
        rm_rows.append(jnp.take_along_axis(rowmax_all[gi], nsel, axis=0))
        c0_rows.append(jnp.take_along_axis(col0_all[gi], nsel, axis=0))
    rowmax_sel = jnp.concatenate(rm_rows, axis=0)
    col0_sel = jnp.concatenate(c0_rows, axis=0)

    midx = jax.lax.broadcasted_iota(jnp.int32, (GK, M), 1)
    nz = (col0_sel < rowmax_sel) & (sel > 0)
    lens = jnp.max(jnp.where(nz, midx + 1, 0), axis=1).reshape(GK, 1)

    rr = jax.lax.broadcasted_iota(jnp.int32, (GK, GK), 0)
    cc = jax.lax.broadcasted_iota(jnp.int32, (GK, GK), 1)
    same_g = (rr // K) == (cc // K)
    lens_row = lens.reshape(1, GK)
    excl = jnp.sum(jnp.where(same_g & (cc < rr),
                             jnp.broadcast_to(lens_row, (GK, GK)), 0),
                   axis=1).reshape(GK, 1)
    idx = jnp.minimum(excl, M)
    olen = jnp.minimum(lens, M - idx)

    j = jax.lax.broadcasted_iota(jnp.int32, (GK, M), 1)
    in_seg = (j >= idx) & (j < idx + olen)
    srcval = (sel - 1) * M + (j - idx)
    masked_src = jnp.where(in_seg, srcval, 0)
    cov = in_seg.astype(jnp.int32)

    for gi in range(G):
        ms_g = masked_src[gi * K:(gi + 1) * K]
        cov_g = cov[gi * K:(gi + 1) * K]
        src = jnp.sum(ms_g, axis=0).reshape(M, 1)
        covered = jnp.sum(cov_g, axis=0).reshape(M, 1) > 0
        src = jnp.where(covered, src, -1)
        i_flat = jax.lax.broadcasted_iota(jnp.int32, (M, N * M), 1)
        P = (i_flat == src).astype(jnp.float32)
        dec_flat = dec_all[gi].reshape(N * M, V)
        out_ref[gi] = jax.lax.dot(P, dec_flat,
                                  preferred_element_type=jnp.float32)


def kernel(decodings, variables, dec_sem_logits, gumbel_noise, target_types, spans):
    del variables
    spans = spans.astype(jnp.int32)
    target_types = target_types.astype(jnp.int32)

    grid_spec = pltpu.PrefetchScalarGridSpec(
        num_scalar_prefetch=2,
        grid=(B // G,),
        in_specs=[
            pl.BlockSpec((N - 1, T - 9, K, N + 1), lambda i, *_: (0, 0, 0, 0)),
            pl.BlockSpec((G, K, N + 1), lambda i, *_: (i, 0, 0)),
            pl.BlockSpec((G, N, M, V), lambda i, *_: (i, 0, 0, 0)),
        ],
        out_specs=pl.BlockSpec((G, M, V), lambda i, *_: (i, 0, 0)),
    )
    return pl.pallas_call(
        _kernel,
        out_shape=jax.ShapeDtypeStruct((B, M, V), jnp.float32),
        grid_spec=grid_spec,
        compiler_params=pltpu.CompilerParams(
            dimension_semantics=("parallel",),
            vmem_limit_bytes=100 * 1024 * 1024,
        ),
        name="tree_lstm_template",
    )(spans, target_types, dec_sem_logits, gumbel_noise, decodings)


# speedup vs baseline: 1.6880x; 1.1669x over previous
import jax
import jax.numpy as jnp
from jax.experimental import pallas as pl
from jax.experimental.pallas import tpu as pltpu

B, N, M, V = 128, 8, 64, 512
K = 8
T = 30
PAD = 0
NEG_INF = -1e30
G = 8
GK = G * K


def _kernel(spans_ref, tt_ref, logits_ref, gumbel_ref, dec_ref, out_ref):
    i = pl.program_id(0)

    logit_rows = []
    span_cols = []
    tt_cols = []
    for gi in range(G):
        b = i * G + gi
        span_b = spans_ref[b]
        tt_b = tt_ref[b]
        logit_rows.append(logits_ref[span_b - 2, tt_b - 9])
        span_cols.append(jnp.full((K, 1), span_b, jnp.int32))
        tt_cols.append(jnp.full((K, 1), tt_b, jnp.int32))
    logits = jnp.concatenate(logit_rows, axis=0)
    span_col = jnp.concatenate(span_cols, axis=0)
    tt_col = jnp.concatenate(tt_cols, axis=0)
    gum = gumbel_ref[:].reshape(GK, N + 1)

    col = jax.lax.broadcasted_iota(jnp.int32, (GK, N + 1), 1)
    masked = jnp.where(col <= span_col, logits, NEG_INF)
    shifted = masked - jnp.max(masked, axis=1, keepdims=True)
    logp = shifted - jnp.log(jnp.sum(jnp.exp(shifted), axis=1, keepdims=True))
    z = logp + gum
    ez = jnp.exp(z - jnp.max(z, axis=1, keepdims=True))
    soft = ez / jnp.sum(ez, axis=1, keepdims=True)
    sel = jnp.argmax(soft, axis=1).reshape(GK, 1).astype(jnp.int32)
    krow = jax.lax.broadcasted_iota(jnp.int32, (GK, 1), 0) % K
    sel = jnp.where(tt_col == 20, jnp.where(krow == 0, 1, 0), sel)

    dec_all = dec_ref[:]
    rowmax_all = jnp.max(dec_all, axis=3)
    col0_all = dec_all[:, :, :, 0]

    rm_rows = []
    c0_rows = []
    for gi in range(G):
        sel_g = sel[gi * K:(gi + 1) * K]
        nsel = jnp.broadcast_to(jnp.maximum(sel_g - 1, 0), (K, M))
        rm_rows.append(jnp.take_along_axis(rowmax_all[gi], nsel, axis=0))
        c0_rows.append(jnp.take_along_axis(col0_all[gi], nsel, axis=0))
    rowmax_sel = jnp.concatenate(rm_rows, axis=0)
    col0_sel = jnp.concatenate(c0_rows, axis=0)

    midx = jax.lax.broadcasted_iota(jnp.int32, (GK, M), 1)
    nz = (col0_sel < rowmax_sel) & (sel > 0)
    lens = jnp.max(jnp.where(nz, midx + 1, 0), axis=1).reshape(GK, 1)

    rr = jax.lax.broadcasted_iota(jnp.int32, (GK, GK), 0)
    cc = jax.lax.broadcasted_iota(jnp.int32, (GK, GK), 1)
    same_g = (rr // K) == (cc // K)
    lens_row = lens.reshape(1, GK)
    excl = jnp.sum(jnp.where(same_g & (cc < rr),
                             jnp.broadcast_to(lens_row, (GK, GK)), 0),
                   axis=1).reshape(GK, 1)
    idx = jnp.minimum(excl, M)
    olen = jnp.minimum(lens, M - idx)

    j = jax.lax.broadcasted_iota(jnp.int32, (GK, M), 1)
    in_seg = (j >= idx) & (j < idx + olen)
    srcval = (sel - 1) * M + (j - idx)
    masked_src = jnp.where(in_seg, srcval, 0)
    cov = in_seg.astype(jnp.int32)

    for gi in range(G):
        ms_g = masked_src[gi * K:(gi + 1) * K]
        cov_g = cov[gi * K:(gi + 1) * K]
        src = jnp.sum(ms_g, axis=0).reshape(M, 1)
        covered = jnp.sum(cov_g, axis=0).reshape(M, 1) > 0
        src = jnp.where(covered, src, -1)
        i_flat = jax.lax.broadcasted_iota(jnp.int32, (M, N * M), 1)
        P = (i_flat == src).astype(jnp.float32)
        dec_flat = dec_all[gi].reshape(N * M, V)
        out_ref[gi] = jax.lax.dot(P, dec_flat,
                                  preferred_element_type=jnp.float32)


def kernel(decodings, variables, dec_sem_logits, gumbel_noise, target_types, spans):
    del variables
    spans = spans.astype(jnp.int32)
    target_types = target_types.astype(jnp.int32)

    grid_spec = pltpu.PrefetchScalarGridSpec(
        num_scalar_prefetch=2,
        grid=(B // G,),
        in_specs=[
            pl.BlockSpec((N - 1, T - 9, K, N + 1), lambda i, *_: (0, 0, 0, 0)),
            pl.BlockSpec((G, K, N + 1), lambda i, *_: (i, 0, 0)),
            pl.BlockSpec((G, N, M, V), lambda i, *_: (i, 0, 0, 0)),
        ],
        out_specs=pl.BlockSpec((G, M, V), lambda i, *_: (i, 0, 0)),
    )
    return pl.pallas_call(
        _kernel,
        out_shape=jax.ShapeDtypeStruct((B, M, V), jnp.float32),
        grid_spec=grid_spec,
        compiler_params=pltpu.CompilerParams(
            dimension_semantics=("parallel",),
            vmem_limit_bytes=100 * 1024 * 1024,
        ),
        name="tree_lstm_template",
    )(spans, target_types, dec_sem_logits, gumbel_noise, decodings)
